# Initial kernel scaffold; baseline (speedup 1.0000x reference)
import jax
import jax.numpy as jnp
from jax.experimental import pallas as pl


def kernel(input, indices):
    raise NotImplementedError("write your pallas kernel here")


# speedup vs baseline: 12.2017x; 12.2017x over previous
import functools

import jax
import jax.numpy as jnp
from jax import lax
from jax.experimental import pallas as pl
from jax.experimental.pallas import tpu as pltpu
from jax.experimental.pallas import tpu_sc as plsc

_H_IN = 4096
_H_OUT = 2 * _H_IN
_ROWS = 4096
_L = 16
_NUM_WORKERS = 32
_ROWS_PER_WORKER = _ROWS // _NUM_WORKERS
_BATCH = 16
_BATCHES = _ROWS_PER_WORKER // _BATCH
_CHUNK_ROWS = 4
_NVREG = _H_IN // _L

_SUBSTAGES = []
_w = 2
while _w <= _H_IN:
    _SUBSTAGES.append(("vr", _w, 0))
    _j = _w // 4
    while _j >= 1:
        _SUBSTAGES.append(("cl", _w, _j))
        _j //= 2
    _w *= 2


def _log2(x):
    return x.bit_length() - 1


def _make_unpool_kernel():
    mesh = plsc.VectorSubcoreMesh(core_axis_name="c", subcore_axis_name="s")

    @functools.partial(
        pl.kernel,
        mesh=mesh,
        out_type=jax.ShapeDtypeStruct((_ROWS * _H_OUT,), jnp.float32),
        scratch_types=[
            pltpu.VMEM((_H_IN * _L,), jnp.int32),
            pltpu.VMEM((_CHUNK_ROWS * _H_IN,), jnp.int32),
            pltpu.VMEM((2 * _H_IN,), jnp.float32),
            pltpu.VMEM((2 * _H_OUT,), jnp.float32),
            pltpu.SemaphoreType.DMA,
            pltpu.SemaphoreType.DMA,
            pltpu.SemaphoreType.DMA,
        ],
        compiler_params=pltpu.CompilerParams(needs_layout_passes=False),
    )
    def unpool(in_hbm, idx_hbm, out_hbm, packed, chunk, valb, outb,
               sem_idx, sem_val, sem_out):
        wid = lax.axis_index("s") * 2 + lax.axis_index("c")
        row0 = wid * _ROWS_PER_WORKER

        iota = lax.iota(jnp.int32, _L)
        iota16 = iota * _L
        zeros = jnp.zeros((_L,), jnp.float32)

        def batch_body(b, carry):
            base = row0 + b * _BATCH

            def chunk_body(ch, c1):
                r0 = base + ch * _CHUNK_ROWS
                pltpu.sync_copy(
                    idx_hbm.at[pl.ds(r0 * _H_IN, _CHUNK_ROWS * _H_IN)],
                    chunk,
                )

                def lrow_body(l, c2):
                    l16 = ch * _CHUNK_ROWS + l

                    def tcol_body(c, c3):
                        a = chunk[pl.ds(l * _H_IN + c * _L, _L)]
                        p = (a << 12) | (iota + c * _L)
                        dst = iota16 + (c * (_L * _L) + l16)
                        plsc.store_scatter(packed, [dst], p)
                        return c3

                    lax.fori_loop(0, _NVREG, tcol_body, 0, unroll=4)
                    return c2

                lax.fori_loop(0, _CHUNK_ROWS, lrow_body, 0)
                return c1

            lax.fori_loop(0, _BATCH // _CHUNK_ROWS, chunk_body, 0)

            for kind, w, j in _SUBSTAGES:
                if kind == "vr":
                    hw = _log2(w) - 1

                    def vr_body(q, c, hw=hw, w=w):
                        blk = (q >> hw) << (hw + 1)
                        off = q & ((w // 2) - 1)
                        i1 = (blk + off) * _L
                        i2 = (blk + (w - 1) - off) * _L
                        a = packed[pl.ds(pl.multiple_of(i1, _L), _L)]
                        bb = packed[pl.ds(pl.multiple_of(i2, _L), _L)]
                        sw = (a >> 12) > (bb >> 12)
                        packed[pl.ds(pl.multiple_of(i1, _L), _L)] = (
                            jnp.where(sw, bb, a))
                        packed[pl.ds(pl.multiple_of(i2, _L), _L)] = (
                            jnp.where(sw, a, bb))
                        return c

                    lax.fori_loop(0, _H_IN // 2, vr_body, 0, unroll=2)
                else:
                    lj = _log2(j)

                    def cl_body(q, c, lj=lj, j=j):
                        i1 = (((q >> lj) << (lj + 1)) + (q & (j - 1))) * _L
                        i2 = i1 + j * _L
                        a = packed[pl.ds(pl.multiple_of(i1, _L), _L)]
                        bb = packed[pl.ds(pl.multiple_of(i2, _L), _L)]
                        sw = (a >> 12) > (bb >> 12)
                        packed[pl.ds(pl.multiple_of(i1, _L), _L)] = (
                            jnp.where(sw, bb, a))
                        packed[pl.ds(pl.multiple_of(i2, _L), _L)] = (
                            jnp.where(sw, a, bb))
                        return c

                    lax.fori_loop(0, _H_IN // 2, cl_body, 0, unroll=2)

            def out_body(l16, c1):
                row = base + l16
                vof = (l16 & 1) * _H_IN
                oof = (l16 & 1) * _H_OUT
                pltpu.sync_copy(in_hbm.at[pl.ds(row * _H_IN, _H_IN)],
                                valb.at[pl.ds(pl.multiple_of(vof, _L),
                                              _H_IN)])

                def zero_body(z, c2):
                    valb
                    outb[pl.ds(pl.multiple_of(oof + z * _L, _L), _L)] = zeros
                    return c2

                lax.fori_loop(0, _H_OUT // _L, zero_body, 0, unroll=4)

                def scat_body(c, c2):
                    src = iota16 + (c * (_L * _L) + l16)
                    x = plsc.load_gather(packed, [src])
                    key = x >> 12
                    pos = x & 0xFFF
                    vals = plsc.load_gather(valb, [pos + vof])
                    plsc.store_scatter(outb, [key + oof], vals)
                    return c2

                lax.fori_loop(0, _NVREG, scat_body, 0, unroll=2)
                pltpu.sync_copy(outb.at[pl.ds(pl.multiple_of(oof, _L),
                                              _H_OUT)],
                                out_hbm.at[pl.ds(row * _H_OUT, _H_OUT)])
                return c1

            lax.fori_loop(0, _BATCH, out_body, 0)
            return carry

        lax.fori_loop(0, _BATCHES, batch_body, 0)

    return unpool


_unpool = _make_unpool_kernel()


def kernel(input, indices):
    N, C, H_in = input.shape
    flat_in = input.reshape(N * C * H_in)
    flat_idx = indices.reshape(N * C * H_in).astype(jnp.int32)
    out = _unpool(flat_in, flat_idx)
    return out.reshape(N, C, _H_OUT)


# speedup vs baseline: 27.5323x; 2.2564x over previous
import functools

import jax
import jax.numpy as jnp
from jax import lax
from jax.experimental import pallas as pl
from jax.experimental.pallas import tpu as pltpu
from jax.experimental.pallas import tpu_sc as plsc

_H_IN = 4096
_H_OUT = 2 * _H_IN
_ROWS = 4096
_L = 16
_NUM_WORKERS = 32
_ROWS_PER_WORKER = _ROWS // _NUM_WORKERS
_BATCH = 16
_BATCHES = _ROWS_PER_WORKER // _BATCH
_CHUNK_ROWS = 4
_NVREG = _H_IN // _L

_SUBSTAGES = []
_w = 2
while _w <= _H_IN:
    _SUBSTAGES.append(("vr", _w, 0))
    _j = _w // 4
    while _j >= 1:
        _SUBSTAGES.append(("cl", _w, _j))
        _j //= 2
    _w *= 2


def _log2(x):
    return x.bit_length() - 1


def _make_unpool_kernel():
    mesh = plsc.VectorSubcoreMesh(core_axis_name="c", subcore_axis_name="s")

    @functools.partial(
        pl.kernel,
        mesh=mesh,
        out_type=jax.ShapeDtypeStruct((_ROWS * _H_OUT,), jnp.float32),
        scratch_types=[
            pltpu.VMEM((_H_IN * _L,), jnp.int32),
            pltpu.VMEM((_CHUNK_ROWS * _H_IN,), jnp.int32),
            pltpu.VMEM((2 * _H_IN,), jnp.float32),
            pltpu.VMEM((2 * _H_OUT,), jnp.float32),
            pltpu.SemaphoreType.DMA,
            pltpu.SemaphoreType.DMA,
            pltpu.SemaphoreType.DMA,
        ],
        compiler_params=pltpu.CompilerParams(needs_layout_passes=False),
    )
    def unpool(in_hbm, idx_hbm, out_hbm, packed, chunk, valb, outb,
               sem_idx, sem_val, sem_out):
        wid = lax.axis_index("s") * 2 + lax.axis_index("c")
        row0 = wid * _ROWS_PER_WORKER

        iota = lax.iota(jnp.int32, _L)
        iota16 = iota * _L
        zeros = jnp.zeros((_L,), jnp.float32)

        def batch_body(b, carry):
            base = row0 + b * _BATCH

            def chunk_body(ch, c1):
                r0 = base + ch * _CHUNK_ROWS
                pltpu.sync_copy(
                    idx_hbm.at[pl.ds(r0 * _H_IN, _CHUNK_ROWS * _H_IN)],
                    chunk,
                )

                def lrow_body(l, c2):
                    l16 = ch * _CHUNK_ROWS + l

                    def tcol_body(c):
                        a = chunk[pl.ds(l * _H_IN + c * _L, _L)]
                        p = (a << 12) | (iota + c * _L)
                        dst = iota16 + (c * (_L * _L) + l16)
                        plsc.store_scatter(packed, [dst], p)

                    plsc.parallel_loop(0, _NVREG, unroll=4)(tcol_body)
                    return c2

                lax.fori_loop(0, _CHUNK_ROWS, lrow_body, 0)
                return c1

            lax.fori_loop(0, _BATCH // _CHUNK_ROWS, chunk_body, 0)

            for kind, w, j in _SUBSTAGES:
                if kind == "vr":
                    hw = _log2(w) - 1

                    def vr_body(q, hw=hw, w=w):
                        blk = (q >> hw) << (hw + 1)
                        off = q & ((w // 2) - 1)
                        i1 = (blk + off) * _L
                        i2 = (blk + (w - 1) - off) * _L
                        a = packed[pl.ds(pl.multiple_of(i1, _L), _L)]
                        bb = packed[pl.ds(pl.multiple_of(i2, _L), _L)]
                        sw = (a >> 12) > (bb >> 12)
                        packed[pl.ds(pl.multiple_of(i1, _L), _L)] = (
                            jnp.where(sw, bb, a))
                        packed[pl.ds(pl.multiple_of(i2, _L), _L)] = (
                            jnp.where(sw, a, bb))

                    plsc.parallel_loop(0, _H_IN // 2, unroll=4)(vr_body)
                else:
                    lj = _log2(j)

                    def cl_body(q, lj=lj, j=j):
                        i1 = (((q >> lj) << (lj + 1)) + (q & (j - 1))) * _L
                        i2 = i1 + j * _L
                        a = packed[pl.ds(pl.multiple_of(i1, _L), _L)]
                        bb = packed[pl.ds(pl.multiple_of(i2, _L), _L)]
                        sw = (a >> 12) > (bb >> 12)
                        packed[pl.ds(pl.multiple_of(i1, _L), _L)] = (
                            jnp.where(sw, bb, a))
                        packed[pl.ds(pl.multiple_of(i2, _L), _L)] = (
                            jnp.where(sw, a, bb))

                    plsc.parallel_loop(0, _H_IN // 2, unroll=4)(cl_body)

            def out_body(l16, c1):
                row = base + l16
                vof = (l16 & 1) * _H_IN
                oof = (l16 & 1) * _H_OUT
                pltpu.sync_copy(in_hbm.at[pl.ds(row * _H_IN, _H_IN)],
                                valb.at[pl.ds(pl.multiple_of(vof, _L),
                                              _H_IN)])

                def zero_body(z):
                    outb[pl.ds(pl.multiple_of(oof + z * _L, _L), _L)] = zeros

                plsc.parallel_loop(0, _H_OUT // _L, unroll=4)(zero_body)

                def scat_body(c, c2):
                    src = iota16 + (c * (_L * _L) + l16)
                    x = plsc.load_gather(packed, [src])
                    key = x >> 12
                    pos = x & 0xFFF
                    vals = plsc.load_gather(valb, [pos + vof])
                    plsc.store_scatter(outb, [key + oof], vals)
                    return c2

                lax.fori_loop(0, _NVREG, scat_body, 0, unroll=2)
                pltpu.sync_copy(outb.at[pl.ds(pl.multiple_of(oof, _L),
                                              _H_OUT)],
                                out_hbm.at[pl.ds(row * _H_OUT, _H_OUT)])
                return c1

            lax.fori_loop(0, _BATCH, out_body, 0)
            return carry

        lax.fori_loop(0, _BATCHES, batch_body, 0)

    return unpool


_unpool = _make_unpool_kernel()


def kernel(input, indices):
    N, C, H_in = input.shape
    flat_in = input.reshape(N * C * H_in)
    flat_idx = indices.reshape(N * C * H_in).astype(jnp.int32)
    out = _unpool(flat_in, flat_idx)
    return out.reshape(N, C, _H_OUT)


# speedup vs baseline: 29.3024x; 1.0643x over previous
import functools

import jax
import jax.numpy as jnp
from jax import lax
from jax.experimental import pallas as pl
from jax.experimental.pallas import tpu as pltpu
from jax.experimental.pallas import tpu_sc as plsc

_H_IN = 4096
_H_OUT = 2 * _H_IN
_ROWS = 4096
_L = 16
_NUM_WORKERS = 32
_ROWS_PER_WORKER = _ROWS // _NUM_WORKERS
_BATCH = 16
_BATCHES = _ROWS_PER_WORKER // _BATCH
_CHUNK_ROWS = 4
_NVREG = _H_IN // _L

_SUBSTAGES = []
_w = 2
while _w <= _H_IN:
    _SUBSTAGES.append(("vr", _w, 0))
    _j = _w // 4
    while _j >= 1:
        _SUBSTAGES.append(("cl", _w, _j))
        _j //= 2
    _w *= 2


def _log2(x):
    return x.bit_length() - 1


def _make_unpool_kernel():
    mesh = plsc.VectorSubcoreMesh(core_axis_name="c", subcore_axis_name="s")

    @functools.partial(
        pl.kernel,
        mesh=mesh,
        out_type=jax.ShapeDtypeStruct((_ROWS * _H_OUT,), jnp.float32),
        scratch_types=[
            pltpu.VMEM((_H_IN * _L,), jnp.int32),
            pltpu.VMEM((_CHUNK_ROWS * _H_IN,), jnp.int32),
            pltpu.VMEM((2 * _H_IN,), jnp.float32),
            pltpu.VMEM((2 * _H_OUT,), jnp.float32),
            pltpu.SemaphoreType.DMA,
            pltpu.SemaphoreType.DMA,
            pltpu.SemaphoreType.DMA,
        ],
        compiler_params=pltpu.CompilerParams(needs_layout_passes=False),
    )
    def unpool(in_hbm, idx_hbm, out_hbm, packed, chunk, valb, outb,
               sem_idx, sem_val, sem_out):
        wid = lax.axis_index("s") * 2 + lax.axis_index("c")
        row0 = wid * _ROWS_PER_WORKER

        iota = lax.iota(jnp.int32, _L)
        iota16 = iota * _L
        zeros = jnp.zeros((_L,), jnp.float32)

        def batch_body(b, carry):
            base = row0 + b * _BATCH

            def chunk_body(ch, c1):
                r0 = base + ch * _CHUNK_ROWS
                pltpu.sync_copy(
                    idx_hbm.at[pl.ds(r0 * _H_IN, _CHUNK_ROWS * _H_IN)],
                    chunk,
                )

                def lrow_body(l, c2):
                    l16 = ch * _CHUNK_ROWS + l

                    def tcol_body(c):
                        a = chunk[pl.ds(l * _H_IN + c * _L, _L)]
                        p = (a << 12) | (iota + c * _L)
                        dst = iota16 + (c * (_L * _L) + l16)
                        plsc.store_scatter(packed, [dst], p)

                    plsc.parallel_loop(0, _NVREG, unroll=4)(tcol_body)
                    return c2

                lax.fori_loop(0, _CHUNK_ROWS, lrow_body, 0)
                return c1

            lax.fori_loop(0, _BATCH // _CHUNK_ROWS, chunk_body, 0)

            for kind, w, j in _SUBSTAGES:
                if kind == "vr":
                    hw = _log2(w) - 1

                    def vr_body(q, hw=hw, w=w):
                        blk = (q >> hw) << (hw + 1)
                        off = q & ((w // 2) - 1)
                        i1 = (blk + off) * _L
                        i2 = (blk + (w - 1) - off) * _L
                        a = packed[pl.ds(pl.multiple_of(i1, _L), _L)]
                        bb = packed[pl.ds(pl.multiple_of(i2, _L), _L)]
                        sw = (a >> 12) > (bb >> 12)
                        packed[pl.ds(pl.multiple_of(i1, _L), _L)] = (
                            jnp.where(sw, bb, a))
                        packed[pl.ds(pl.multiple_of(i2, _L), _L)] = (
                            jnp.where(sw, a, bb))

                    plsc.parallel_loop(0, _H_IN // 2, unroll=8)(vr_body)
                else:
                    lj = _log2(j)

                    def cl_body(q, lj=lj, j=j):
                        i1 = (((q >> lj) << (lj + 1)) + (q & (j - 1))) * _L
                        i2 = i1 + j * _L
                        a = packed[pl.ds(pl.multiple_of(i1, _L), _L)]
                        bb = packed[pl.ds(pl.multiple_of(i2, _L), _L)]
                        sw = (a >> 12) > (bb >> 12)
                        packed[pl.ds(pl.multiple_of(i1, _L), _L)] = (
                            jnp.where(sw, bb, a))
                        packed[pl.ds(pl.multiple_of(i2, _L), _L)] = (
                            jnp.where(sw, a, bb))

                    plsc.parallel_loop(0, _H_IN // 2, unroll=8)(cl_body)

            def out_body(l16, c1):
                row = base + l16
                vof = (l16 & 1) * _H_IN
                oof = (l16 & 1) * _H_OUT
                pltpu.sync_copy(in_hbm.at[pl.ds(row * _H_IN, _H_IN)],
                                valb.at[pl.ds(pl.multiple_of(vof, _L),
                                              _H_IN)])

                def zero_body(z):
                    outb[pl.ds(pl.multiple_of(oof + z * _L, _L), _L)] = zeros

                plsc.parallel_loop(0, _H_OUT // _L, unroll=4)(zero_body)

                def scat_body(c, c2):
                    src = iota16 + (c * (_L * _L) + l16)
                    x = plsc.load_gather(packed, [src])
                    key = x >> 12
                    pos = x & 0xFFF
                    vals = plsc.load_gather(valb, [pos + vof])
                    plsc.store_scatter(outb, [key + oof], vals)
                    return c2

                lax.fori_loop(0, _NVREG, scat_body, 0, unroll=2)
                pltpu.sync_copy(outb.at[pl.ds(pl.multiple_of(oof, _L),
                                              _H_OUT)],
                                out_hbm.at[pl.ds(row * _H_OUT, _H_OUT)])
                return c1

            lax.fori_loop(0, _BATCH, out_body, 0)
            return carry

        lax.fori_loop(0, _BATCHES, batch_body, 0)

    return unpool


_unpool = _make_unpool_kernel()


def kernel(input, indices):
    N, C, H_in = input.shape
    flat_in = input.reshape(N * C * H_in)
    flat_idx = indices.reshape(N * C * H_in).astype(jnp.int32)
    out = _unpool(flat_in, flat_idx)
    return out.reshape(N, C, _H_OUT)


# speedup vs baseline: 34.6311x; 1.1819x over previous
import functools

import jax
import jax.numpy as jnp
from jax import lax
from jax.experimental import pallas as pl
from jax.experimental.pallas import tpu as pltpu
from jax.experimental.pallas import tpu_sc as plsc

_H_IN = 4096
_H_OUT = 2 * _H_IN
_ROWS = 4096
_L = 16
_NUM_WORKERS = 32
_ROWS_PER_WORKER = _ROWS // _NUM_WORKERS
_BATCH = 16
_BATCHES = _ROWS_PER_WORKER // _BATCH
_CHUNK_ROWS = 4
_NVREG = _H_IN // _L

_SUBSTAGES = []
_w = 2
while _w <= _H_IN:
    _SUBSTAGES.append(("vr", _w, 0))
    _j = _w // 4
    while _j >= 1:
        _SUBSTAGES.append(("cl", _w, _j))
        _j //= 2
    _w *= 2


def _log2(x):
    return x.bit_length() - 1


def _make_unpool_kernel():
    mesh = plsc.VectorSubcoreMesh(core_axis_name="c", subcore_axis_name="s")

    @functools.partial(
        pl.kernel,
        mesh=mesh,
        out_type=jax.ShapeDtypeStruct((_ROWS * _H_OUT,), jnp.float32),
        scratch_types=[
            pltpu.VMEM((_H_IN * _L + _L,), jnp.int32),
            pltpu.VMEM((2 * _CHUNK_ROWS * _H_IN,), jnp.int32),
            pltpu.VMEM((2 * _H_IN,), jnp.float32),
            pltpu.VMEM((2 * _H_OUT,), jnp.float32),
            pltpu.SemaphoreType.DMA,
            pltpu.SemaphoreType.DMA,
            pltpu.SemaphoreType.DMA,
        ],
        compiler_params=pltpu.CompilerParams(needs_layout_passes=False),
    )
    def unpool(in_hbm, idx_hbm, out_hbm, packed, chunk, valb, outb,
               sem_idx, sem_val, sem_out):
        wid = lax.axis_index("s") * 2 + lax.axis_index("c")
        row0 = wid * _ROWS_PER_WORKER

        iota = lax.iota(jnp.int32, _L)
        iota16 = iota * _L
        zeros = jnp.zeros((_L,), jnp.float32)
        _CHLEN = _CHUNK_ROWS * _H_IN
        _NCHUNK = _BATCH // _CHUNK_ROWS

        packed[pl.ds(_H_IN * _L, _L)] = jnp.full((_L,), 1 << 25, jnp.int32)

        def batch_body(b, carry):
            base = row0 + b * _BATCH

            pltpu.async_copy(idx_hbm.at[pl.ds(base * _H_IN, _CHLEN)],
                             chunk.at[pl.ds(0, _CHLEN)], sem_idx)

            def chunk_body(ch, c1):
                r0 = base + ch * _CHUNK_ROWS
                cof = (ch & 1) * _CHLEN
                pltpu.make_async_copy(
                    idx_hbm.at[pl.ds(r0 * _H_IN, _CHLEN)],
                    chunk.at[pl.ds(pl.multiple_of(cof, _L), _CHLEN)],
                    sem_idx).wait()

                @pl.when(ch + 1 < _NCHUNK)
                def _():
                    pltpu.async_copy(
                        idx_hbm.at[pl.ds((r0 + _CHUNK_ROWS) * _H_IN, _CHLEN)],
                        chunk.at[pl.ds(pl.multiple_of(
                            ((ch + 1) & 1) * _CHLEN, _L), _CHLEN)],
                        sem_idx)

                def lrow_body(l, c2):
                    l16 = ch * _CHUNK_ROWS + l

                    def tcol_body(c):
                        a = chunk[pl.ds(cof + l * _H_IN + c * _L, _L)]
                        p = (a << 12) | (iota + c * _L)
                        dst = iota16 + (c * (_L * _L) + l16)
                        plsc.store_scatter(packed, [dst], p)

                    plsc.parallel_loop(0, _NVREG, unroll=4)(tcol_body)
                    return c2

                lax.fori_loop(0, _CHUNK_ROWS, lrow_body, 0)
                return c1

            lax.fori_loop(0, _NCHUNK, chunk_body, 0)

            for kind, w, j in _SUBSTAGES:
                if kind == "vr":
                    hw = _log2(w) - 1

                    def vr_body(q, hw=hw, w=w):
                        blk = (q >> hw) << (hw + 1)
                        off = q & ((w // 2) - 1)
                        i1 = (blk + off) * _L
                        i2 = (blk + (w - 1) - off) * _L
                        a = packed[pl.ds(pl.multiple_of(i1, _L), _L)]
                        bb = packed[pl.ds(pl.multiple_of(i2, _L), _L)]
                        sw = (a >> 12) > (bb >> 12)
                        packed[pl.ds(pl.multiple_of(i1, _L), _L)] = (
                            jnp.where(sw, bb, a))
                        packed[pl.ds(pl.multiple_of(i2, _L), _L)] = (
                            jnp.where(sw, a, bb))

                    plsc.parallel_loop(0, _H_IN // 2, unroll=8)(vr_body)
                else:
                    lj = _log2(j)

                    def cl_body(q, lj=lj, j=j):
                        i1 = (((q >> lj) << (lj + 1)) + (q & (j - 1))) * _L
                        i2 = i1 + j * _L
                        a = packed[pl.ds(pl.multiple_of(i1, _L), _L)]
                        bb = packed[pl.ds(pl.multiple_of(i2, _L), _L)]
                        sw = (a >> 12) > (bb >> 12)
                        packed[pl.ds(pl.multiple_of(i1, _L), _L)] = (
                            jnp.where(sw, bb, a))
                        packed[pl.ds(pl.multiple_of(i2, _L), _L)] = (
                            jnp.where(sw, a, bb))

                    plsc.parallel_loop(0, _H_IN // 2, unroll=8)(cl_body)

            pltpu.async_copy(in_hbm.at[pl.ds(base * _H_IN, _H_IN)],
                             valb.at[pl.ds(0, _H_IN)], sem_val)

            def out_body(l16, c1):
                row = base + l16
                sl = l16 & 1
                vof = sl * _H_IN
                oof = sl * _H_OUT

                @pl.when(l16 >= 2)
                def _():
                    pltpu.make_async_copy(
                        outb.at[pl.ds(pl.multiple_of(oof, _L), _H_OUT)],
                        out_hbm.at[pl.ds((row - 2) * _H_OUT, _H_OUT)],
                        sem_out).wait()

                @pl.when(l16 + 1 < _BATCH)
                def _():
                    pltpu.async_copy(
                        in_hbm.at[pl.ds((row + 1) * _H_IN, _H_IN)],
                        valb.at[pl.ds(pl.multiple_of((1 - sl) * _H_IN, _L),
                                      _H_IN)],
                        sem_val)

                def zero_body(z):
                    outb[pl.ds(pl.multiple_of(oof + z * _L, _L), _L)] = zeros

                plsc.parallel_loop(0, _H_OUT // _L, unroll=4)(zero_body)

                pltpu.make_async_copy(
                    in_hbm.at[pl.ds(row * _H_IN, _H_IN)],
                    valb.at[pl.ds(pl.multiple_of(vof, _L), _H_IN)],
                    sem_val).wait()

                def scat_body(c):
                    src = iota16 + (c * (_L * _L) + l16)
                    x = plsc.load_gather(packed, [src])
                    nx = plsc.load_gather(packed, [src + _L])
                    key = x >> 12
                    keep = key != (nx >> 12)
                    pos = x & 0xFFF
                    vals = plsc.load_gather(valb, [pos + vof])
                    plsc.store_scatter(outb, [key + oof], vals, mask=keep)

                plsc.parallel_loop(0, _NVREG, unroll=4)(scat_body)
                pltpu.async_copy(
                    outb.at[pl.ds(pl.multiple_of(oof, _L), _H_OUT)],
                    out_hbm.at[pl.ds(row * _H_OUT, _H_OUT)], sem_out)
                return c1

            lax.fori_loop(0, _BATCH, out_body, 0)
            pltpu.make_async_copy(
                outb.at[pl.ds(0, _H_OUT)],
                out_hbm.at[pl.ds((base + _BATCH - 2) * _H_OUT, _H_OUT)],
                sem_out).wait()
            pltpu.make_async_copy(
                outb.at[pl.ds(_H_OUT, _H_OUT)],
                out_hbm.at[pl.ds((base + _BATCH - 1) * _H_OUT, _H_OUT)],
                sem_out).wait()
            return carry

        lax.fori_loop(0, _BATCHES, batch_body, 0)

    return unpool


_unpool = _make_unpool_kernel()


def kernel(input, indices):
    N, C, H_in = input.shape
    flat_in = input.reshape(N * C * H_in)
    flat_idx = indices.reshape(N * C * H_in).astype(jnp.int32)
    out = _unpool(flat_in, flat_idx)
    return out.reshape(N, C, _H_OUT)


# speedup vs baseline: 38.1377x; 1.1013x over previous
import functools

import jax
import jax.numpy as jnp
from jax import lax
from jax.experimental import pallas as pl
from jax.experimental.pallas import tpu as pltpu
from jax.experimental.pallas import tpu_sc as plsc

_H_IN = 4096
_H_OUT = 2 * _H_IN
_ROWS = 4096
_L = 16
_NUM_WORKERS = 32
_ROWS_PER_WORKER = _ROWS // _NUM_WORKERS
_BATCH = 16
_BATCHES = _ROWS_PER_WORKER // _BATCH
_CHUNK_ROWS = 4
_NVREG = _H_IN // _L

_PASSES = [("j1",)]
_w = 4
while _w <= _H_IN:
    _PASSES.append(("vrc", _w))
    _rem = _w // 8
    while _rem >= 2:
        _PASSES.append(("clp", _rem))
        _rem //= 4
    if _rem == 1:
        _PASSES.append(("j1",))
    _w *= 2


def _log2(x):
    return x.bit_length() - 1


def _make_unpool_kernel():
    mesh = plsc.VectorSubcoreMesh(core_axis_name="c", subcore_axis_name="s")

    @functools.partial(
        pl.kernel,
        mesh=mesh,
        out_type=jax.ShapeDtypeStruct((_ROWS * _H_OUT,), jnp.float32),
        scratch_types=[
            pltpu.VMEM((_H_IN * _L + _L,), jnp.int32),
            pltpu.VMEM((2 * _CHUNK_ROWS * _H_IN,), jnp.int32),
            pltpu.VMEM((2 * _H_IN,), jnp.float32),
            pltpu.VMEM((2 * _H_OUT,), jnp.float32),
            pltpu.SemaphoreType.DMA,
            pltpu.SemaphoreType.DMA,
            pltpu.SemaphoreType.DMA,
        ],
        compiler_params=pltpu.CompilerParams(needs_layout_passes=False),
    )
    def unpool(in_hbm, idx_hbm, out_hbm, packed, chunk, valb, outb,
               sem_idx, sem_val, sem_out):
        wid = lax.axis_index("s") * 2 + lax.axis_index("c")
        row0 = wid * _ROWS_PER_WORKER

        iota = lax.iota(jnp.int32, _L)
        iota16 = iota * _L
        zeros = jnp.zeros((_L,), jnp.float32)
        _CHLEN = _CHUNK_ROWS * _H_IN
        _NCHUNK = _BATCH // _CHUNK_ROWS

        packed[pl.ds(_H_IN * _L, _L)] = jnp.full((_L,), 1 << 25, jnp.int32)

        def batch_body(b, carry):
            base = row0 + b * _BATCH

            pltpu.async_copy(idx_hbm.at[pl.ds(base * _H_IN, _CHLEN)],
                             chunk.at[pl.ds(0, _CHLEN)], sem_idx)

            def chunk_body(ch, c1):
                r0 = base + ch * _CHUNK_ROWS
                cof = (ch & 1) * _CHLEN
                pltpu.make_async_copy(
                    idx_hbm.at[pl.ds(r0 * _H_IN, _CHLEN)],
                    chunk.at[pl.ds(pl.multiple_of(cof, _L), _CHLEN)],
                    sem_idx).wait()

                @pl.when(ch + 1 < _NCHUNK)
                def _():
                    pltpu.async_copy(
                        idx_hbm.at[pl.ds((r0 + _CHUNK_ROWS) * _H_IN, _CHLEN)],
                        chunk.at[pl.ds(pl.multiple_of(
                            ((ch + 1) & 1) * _CHLEN, _L), _CHLEN)],
                        sem_idx)

                def lrow_body(l, c2):
                    l16 = ch * _CHUNK_ROWS + l

                    def tcol_body(c):
                        a = chunk[pl.ds(cof + l * _H_IN + c * _L, _L)]
                        p = (a << 12) | (iota + c * _L)
                        dst = iota16 + (c * (_L * _L) + l16)
                        plsc.store_scatter(packed, [dst], p)

                    plsc.parallel_loop(0, _NVREG, unroll=4)(tcol_body)
                    return c2

                lax.fori_loop(0, _CHUNK_ROWS, lrow_body, 0)
                return c1

            lax.fori_loop(0, _NCHUNK, chunk_body, 0)

            def _ld(i):
                return packed[pl.ds(pl.multiple_of(i, _L), _L)]

            def _st(i, v):
                packed[pl.ds(pl.multiple_of(i, _L), _L)] = v

            def _ce(a, b):
                sw = (a >> 12) > (b >> 12)
                return jnp.where(sw, b, a), jnp.where(sw, a, b)

            for pass_ in _PASSES:
                if pass_[0] == "j1":

                    def j1_body(g):
                        i0 = g * (4 * _L)
                        v0 = _ld(i0)
                        v1 = _ld(i0 + _L)
                        v2 = _ld(i0 + 2 * _L)
                        v3 = _ld(i0 + 3 * _L)
                        v0, v1 = _ce(v0, v1)
                        v2, v3 = _ce(v2, v3)
                        _st(i0, v0)
                        _st(i0 + _L, v1)
                        _st(i0 + 2 * _L, v2)
                        _st(i0 + 3 * _L, v3)

                    plsc.parallel_loop(0, _H_IN // 4, unroll=4)(j1_body)
                elif pass_[0] == "vrc":
                    w = pass_[1]
                    lw = _log2(w)
                    nb = w // 4

                    def vrc_body(g, lw=lw, w=w, nb=nb):
                        blk = (g >> (lw - 2)) << lw
                        off = g & (nb - 1)
                        ia = (blk + off) * _L
                        ib = ia + nb * _L
                        id_ = (blk + (w - 1) - off) * _L
                        ic = id_ - nb * _L
                        va = _ld(ia)
                        vb = _ld(ib)
                        vc = _ld(ic)
                        vd = _ld(id_)
                        va, vd = _ce(va, vd)
                        vb, vc = _ce(vb, vc)
                        va, vb = _ce(va, vb)
                        vc, vd = _ce(vc, vd)
                        _st(ia, va)
                        _st(ib, vb)
                        _st(ic, vc)
                        _st(id_, vd)

                    plsc.parallel_loop(0, _H_IN // 4, unroll=4)(vrc_body)
                else:
                    j1 = pass_[1]
                    j2 = j1 // 2
                    lj1 = _log2(j1)

                    def clp_body(g, lj1=lj1, j1=j1, j2=j2):
                        blk = (g >> (lj1 - 1)) << (lj1 + 1)
                        off = g & (j2 - 1)
                        i0 = (blk + off) * _L
                        i1 = i0 + j2 * _L
                        i2 = i0 + j1 * _L
                        i3 = i2 + j2 * _L
                        v0 = _ld(i0)
                        v1 = _ld(i1)
                        v2 = _ld(i2)
                        v3 = _ld(i3)
                        v0, v2 = _ce(v0, v2)
                        v1, v3 = _ce(v1, v3)
                        v0, v1 = _ce(v0, v1)
                        v2, v3 = _ce(v2, v3)
                        _st(i0, v0)
                        _st(i1, v1)
                        _st(i2, v2)
                        _st(i3, v3)

                    plsc.parallel_loop(0, _H_IN // 4, unroll=4)(clp_body)

            pltpu.async_copy(in_hbm.at[pl.ds(base * _H_IN, _H_IN)],
                             valb.at[pl.ds(0, _H_IN)], sem_val)

            def out_body(l16, c1):
                row = base + l16
                sl = l16 & 1
                vof = sl * _H_IN
                oof = sl * _H_OUT

                @pl.when(l16 >= 2)
                def _():
                    pltpu.make_async_copy(
                        outb.at[pl.ds(pl.multiple_of(oof, _L), _H_OUT)],
                        out_hbm.at[pl.ds((row - 2) * _H_OUT, _H_OUT)],
                        sem_out).wait()

                @pl.when(l16 + 1 < _BATCH)
                def _():
                    pltpu.async_copy(
                        in_hbm.at[pl.ds((row + 1) * _H_IN, _H_IN)],
                        valb.at[pl.ds(pl.multiple_of((1 - sl) * _H_IN, _L),
                                      _H_IN)],
                        sem_val)

                def zero_body(z):
                    outb[pl.ds(pl.multiple_of(oof + z * _L, _L), _L)] = zeros

                plsc.parallel_loop(0, _H_OUT // _L, unroll=4)(zero_body)

                pltpu.make_async_copy(
                    in_hbm.at[pl.ds(row * _H_IN, _H_IN)],
                    valb.at[pl.ds(pl.multiple_of(vof, _L), _H_IN)],
                    sem_val).wait()

                def scat_body(c):
                    src = iota16 + (c * (_L * _L) + l16)
                    x = plsc.load_gather(packed, [src])
                    nx = plsc.load_gather(packed, [src + _L])
                    key = x >> 12
                    keep = key != (nx >> 12)
                    pos = x & 0xFFF
                    vals = plsc.load_gather(valb, [pos + vof])
                    plsc.store_scatter(outb, [key + oof], vals, mask=keep)

                plsc.parallel_loop(0, _NVREG, unroll=4)(scat_body)
                pltpu.async_copy(
                    outb.at[pl.ds(pl.multiple_of(oof, _L), _H_OUT)],
                    out_hbm.at[pl.ds(row * _H_OUT, _H_OUT)], sem_out)
                return c1

            lax.fori_loop(0, _BATCH, out_body, 0)
            pltpu.make_async_copy(
                outb.at[pl.ds(0, _H_OUT)],
                out_hbm.at[pl.ds((base + _BATCH - 2) * _H_OUT, _H_OUT)],
                sem_out).wait()
            pltpu.make_async_copy(
                outb.at[pl.ds(_H_OUT, _H_OUT)],
                out_hbm.at[pl.ds((base + _BATCH - 1) * _H_OUT, _H_OUT)],
                sem_out).wait()
            return carry

        lax.fori_loop(0, _BATCHES, batch_body, 0)

    return unpool


_unpool = _make_unpool_kernel()


def kernel(input, indices):
    N, C, H_in = input.shape
    flat_in = input.reshape(N * C * H_in)
    flat_idx = indices.reshape(N * C * H_in).astype(jnp.int32)
    out = _unpool(flat_in, flat_idx)
    return out.reshape(N, C, _H_OUT)


# speedup vs baseline: 38.4742x; 1.0088x over previous
import functools

import jax
import jax.numpy as jnp
from jax import lax
from jax.experimental import pallas as pl
from jax.experimental.pallas import tpu as pltpu
from jax.experimental.pallas import tpu_sc as plsc

_H_IN = 4096
_H_OUT = 2 * _H_IN
_ROWS = 4096
_L = 16
_NUM_WORKERS = 32
_ROWS_PER_WORKER = _ROWS // _NUM_WORKERS
_BATCH = 16
_BATCHES = _ROWS_PER_WORKER // _BATCH
_CHUNK_ROWS = 4
_NVREG = _H_IN // _L

_PASSES = [("j1",), ("vrc", 4), ("vrc", 8), ("j1",)]
_w = 16
while _w <= _H_IN:
    _PASSES.append(("vrc8", _w))
    _rem = _w // 16
    while _rem >= 4:
        _PASSES.append(("clp3", _rem))
        _rem //= 8
    if _rem == 2:
        _PASSES.append(("clp", 2))
    elif _rem == 1:
        _PASSES.append(("j1",))
    _w *= 2


def _log2(x):
    return x.bit_length() - 1


def _make_unpool_kernel():
    mesh = plsc.VectorSubcoreMesh(core_axis_name="c", subcore_axis_name="s")

    @functools.partial(
        pl.kernel,
        mesh=mesh,
        out_type=jax.ShapeDtypeStruct((_ROWS * _H_OUT,), jnp.float32),
        scratch_types=[
            pltpu.VMEM((_H_IN * _L + _L,), jnp.int32),
            pltpu.VMEM((2 * _CHUNK_ROWS * _H_IN,), jnp.int32),
            pltpu.VMEM((2 * _H_IN,), jnp.float32),
            pltpu.VMEM((2 * _H_OUT,), jnp.float32),
            pltpu.SemaphoreType.DMA,
            pltpu.SemaphoreType.DMA,
            pltpu.SemaphoreType.DMA,
        ],
        compiler_params=pltpu.CompilerParams(needs_layout_passes=False),
    )
    def unpool(in_hbm, idx_hbm, out_hbm, packed, chunk, valb, outb,
               sem_idx, sem_val, sem_out):
        wid = lax.axis_index("s") * 2 + lax.axis_index("c")
        row0 = wid * _ROWS_PER_WORKER

        iota = lax.iota(jnp.int32, _L)
        iota16 = iota * _L
        zeros = jnp.zeros((_L,), jnp.float32)
        _CHLEN = _CHUNK_ROWS * _H_IN
        _NCHUNK = _BATCH // _CHUNK_ROWS

        packed[pl.ds(_H_IN * _L, _L)] = jnp.full((_L,), 1 << 25, jnp.int32)

        def batch_body(b, carry):
            base = row0 + b * _BATCH

            pltpu.async_copy(idx_hbm.at[pl.ds(base * _H_IN, _CHLEN)],
                             chunk.at[pl.ds(0, _CHLEN)], sem_idx)

            def chunk_body(ch, c1):
                r0 = base + ch * _CHUNK_ROWS
                cof = (ch & 1) * _CHLEN
                pltpu.make_async_copy(
                    idx_hbm.at[pl.ds(r0 * _H_IN, _CHLEN)],
                    chunk.at[pl.ds(pl.multiple_of(cof, _L), _CHLEN)],
                    sem_idx).wait()

                @pl.when(ch + 1 < _NCHUNK)
                def _():
                    pltpu.async_copy(
                        idx_hbm.at[pl.ds((r0 + _CHUNK_ROWS) * _H_IN, _CHLEN)],
                        chunk.at[pl.ds(pl.multiple_of(
                            ((ch + 1) & 1) * _CHLEN, _L), _CHLEN)],
                        sem_idx)

                def lrow_body(l, c2):
                    l16 = ch * _CHUNK_ROWS + l

                    def tcol_body(c):
                        a = chunk[pl.ds(cof + l * _H_IN + c * _L, _L)]
                        p = (a << 12) | (iota + c * _L)
                        dst = iota16 + (c * (_L * _L) + l16)
                        plsc.store_scatter(packed, [dst], p)

                    plsc.parallel_loop(0, _NVREG, unroll=4)(tcol_body)
                    return c2

                lax.fori_loop(0, _CHUNK_ROWS, lrow_body, 0)
                return c1

            lax.fori_loop(0, _NCHUNK, chunk_body, 0)

            def _ld(i):
                return packed[pl.ds(pl.multiple_of(i, _L), _L)]

            def _st(i, v):
                packed[pl.ds(pl.multiple_of(i, _L), _L)] = v

            def _ce(a, b):
                sw = (a >> 12) > (b >> 12)
                return jnp.where(sw, b, a), jnp.where(sw, a, b)

            for pass_ in _PASSES:
                if pass_[0] == "j1":

                    def j1_body(g):
                        i0 = g * (4 * _L)
                        v0 = _ld(i0)
                        v1 = _ld(i0 + _L)
                        v2 = _ld(i0 + 2 * _L)
                        v3 = _ld(i0 + 3 * _L)
                        v0, v1 = _ce(v0, v1)
                        v2, v3 = _ce(v2, v3)
                        _st(i0, v0)
                        _st(i0 + _L, v1)
                        _st(i0 + 2 * _L, v2)
                        _st(i0 + 3 * _L, v3)

                    plsc.parallel_loop(0, _H_IN // 4, unroll=4)(j1_body)
                elif pass_[0] == "vrc":
                    w = pass_[1]
                    lw = _log2(w)
                    nb = w // 4

                    def vrc_body(g, lw=lw, w=w, nb=nb):
                        blk = (g >> (lw - 2)) << lw
                        off = g & (nb - 1)
                        ia = (blk + off) * _L
                        ib = ia + nb * _L
                        id_ = (blk + (w - 1) - off) * _L
                        ic = id_ - nb * _L
                        va = _ld(ia)
                        vb = _ld(ib)
                        vc = _ld(ic)
                        vd = _ld(id_)
                        va, vd = _ce(va, vd)
                        vb, vc = _ce(vb, vc)
                        va, vb = _ce(va, vb)
                        vc, vd = _ce(vc, vd)
                        _st(ia, va)
                        _st(ib, vb)
                        _st(ic, vc)
                        _st(id_, vd)

                    plsc.parallel_loop(0, _H_IN // 4, unroll=4)(vrc_body)
                elif pass_[0] == "vrc8":
                    w = pass_[1]
                    lw = _log2(w)
                    ne = w // 8

                    def vrc8_body(g, lw=lw, w=w, ne=ne):
                        blk = (g >> (lw - 3)) << lw
                        off = g & (ne - 1)
                        ii = [(blk + off + k * ne) * _L for k in range(4)]
                        mm = [(blk + (w - 1) - off - k * ne) * _L
                              for k in range(4)]
                        a = [_ld(i) for i in ii]
                        m = [_ld(i) for i in mm]
                        for k in range(4):
                            a[k], m[k] = _ce(a[k], m[k])
                        a[0], a[2] = _ce(a[0], a[2])
                        a[1], a[3] = _ce(a[1], a[3])
                        m[3], m[1] = _ce(m[3], m[1])
                        m[2], m[0] = _ce(m[2], m[0])
                        a[0], a[1] = _ce(a[0], a[1])
                        a[2], a[3] = _ce(a[2], a[3])
                        m[3], m[2] = _ce(m[3], m[2])
                        m[1], m[0] = _ce(m[1], m[0])
                        for k in range(4):
                            _st(ii[k], a[k])
                            _st(mm[k], m[k])

                    plsc.parallel_loop(0, _H_IN // 8, unroll=2)(vrc8_body)
                elif pass_[0] == "clp3":
                    j1 = pass_[1]
                    u = j1 // 4
                    lj1 = _log2(j1)

                    def clp3_body(g, lj1=lj1, u=u):
                        blk = (g >> (lj1 - 2)) << (lj1 + 1)
                        off = g & (u - 1)
                        ii = [(blk + off + k * u) * _L for k in range(8)]
                        v = [_ld(i) for i in ii]
                        v[0], v[4] = _ce(v[0], v[4])
                        v[1], v[5] = _ce(v[1], v[5])
                        v[2], v[6] = _ce(v[2], v[6])
                        v[3], v[7] = _ce(v[3], v[7])
                        v[0], v[2] = _ce(v[0], v[2])
                        v[1], v[3] = _ce(v[1], v[3])
                        v[4], v[6] = _ce(v[4], v[6])
                        v[5], v[7] = _ce(v[5], v[7])
                        v[0], v[1] = _ce(v[0], v[1])
                        v[2], v[3] = _ce(v[2], v[3])
                        v[4], v[5] = _ce(v[4], v[5])
                        v[6], v[7] = _ce(v[6], v[7])
                        for k in range(8):
                            _st(ii[k], v[k])

                    plsc.parallel_loop(0, _H_IN // 8, unroll=2)(clp3_body)
                else:
                    j1 = pass_[1]
                    j2 = j1 // 2
                    lj1 = _log2(j1)

                    def clp_body(g, lj1=lj1, j1=j1, j2=j2):
                        blk = (g >> (lj1 - 1)) << (lj1 + 1)
                        off = g & (j2 - 1)
                        i0 = (blk + off) * _L
                        i1 = i0 + j2 * _L
                        i2 = i0 + j1 * _L
                        i3 = i2 + j2 * _L
                        v0 = _ld(i0)
                        v1 = _ld(i1)
                        v2 = _ld(i2)
                        v3 = _ld(i3)
                        v0, v2 = _ce(v0, v2)
                        v1, v3 = _ce(v1, v3)
                        v0, v1 = _ce(v0, v1)
                        v2, v3 = _ce(v2, v3)
                        _st(i0, v0)
                        _st(i1, v1)
                        _st(i2, v2)
                        _st(i3, v3)

                    plsc.parallel_loop(0, _H_IN // 4, unroll=4)(clp_body)

            pltpu.async_copy(in_hbm.at[pl.ds(base * _H_IN, _H_IN)],
                             valb.at[pl.ds(0, _H_IN)], sem_val)

            def out_body(l16, c1):
                row = base + l16
                sl = l16 & 1
                vof = sl * _H_IN
                oof = sl * _H_OUT

                @pl.when(l16 >= 2)
                def _():
                    pltpu.make_async_copy(
                        outb.at[pl.ds(pl.multiple_of(oof, _L), _H_OUT)],
                        out_hbm.at[pl.ds((row - 2) * _H_OUT, _H_OUT)],
                        sem_out).wait()

                @pl.when(l16 + 1 < _BATCH)
                def _():
                    pltpu.async_copy(
                        in_hbm.at[pl.ds((row + 1) * _H_IN, _H_IN)],
                        valb.at[pl.ds(pl.multiple_of((1 - sl) * _H_IN, _L),
                                      _H_IN)],
                        sem_val)

                def zero_body(z):
                    outb[pl.ds(pl.multiple_of(oof + z * _L, _L), _L)] = zeros

                plsc.parallel_loop(0, _H_OUT // _L, unroll=4)(zero_body)

                pltpu.make_async_copy(
                    in_hbm.at[pl.ds(row * _H_IN, _H_IN)],
                    valb.at[pl.ds(pl.multiple_of(vof, _L), _H_IN)],
                    sem_val).wait()

                def scat_body(c):
                    src = iota16 + (c * (_L * _L) + l16)
                    x = plsc.load_gather(packed, [src])
                    nx = plsc.load_gather(packed, [src + _L])
                    key = x >> 12
                    keep = key != (nx >> 12)
                    pos = x & 0xFFF
                    vals = plsc.load_gather(valb, [pos + vof])
                    plsc.store_scatter(outb, [key + oof], vals, mask=keep)

                plsc.parallel_loop(0, _NVREG, unroll=4)(scat_body)
                pltpu.async_copy(
                    outb.at[pl.ds(pl.multiple_of(oof, _L), _H_OUT)],
                    out_hbm.at[pl.ds(row * _H_OUT, _H_OUT)], sem_out)
                return c1

            lax.fori_loop(0, _BATCH, out_body, 0)
            pltpu.make_async_copy(
                outb.at[pl.ds(0, _H_OUT)],
                out_hbm.at[pl.ds((base + _BATCH - 2) * _H_OUT, _H_OUT)],
                sem_out).wait()
            pltpu.make_async_copy(
                outb.at[pl.ds(_H_OUT, _H_OUT)],
                out_hbm.at[pl.ds((base + _BATCH - 1) * _H_OUT, _H_OUT)],
                sem_out).wait()
            return carry

        lax.fori_loop(0, _BATCHES, batch_body, 0)

    return unpool


_unpool = _make_unpool_kernel()


def kernel(input, indices):
    N, C, H_in = input.shape
    flat_in = input.reshape(N * C * H_in)
    flat_idx = indices.reshape(N * C * H_in).astype(jnp.int32)
    out = _unpool(flat_in, flat_idx)
    return out.reshape(N, C, _H_OUT)


# speedup vs baseline: 43.2807x; 1.1249x over previous
import functools

import jax
import jax.numpy as jnp
from jax import lax
from jax.experimental import pallas as pl
from jax.experimental.pallas import tpu as pltpu
from jax.experimental.pallas import tpu_sc as plsc

_H_IN = 4096
_H_OUT = 2 * _H_IN
_ROWS = 4096
_L = 16
_NUM_WORKERS = 32
_ROWS_PER_WORKER = _ROWS // _NUM_WORKERS
_BATCH = 16
_BATCHES = _ROWS_PER_WORKER // _BATCH
_CHUNK_ROWS = 4
_NVREG = _H_IN // _L

_PASSES = [("j1",), ("vrc", 4), ("vrc", 8), ("j1",)]
_w = 16
while _w <= _H_IN:
    _PASSES.append(("vrc8", _w))
    _rem = _w // 16
    while _rem >= 4:
        _PASSES.append(("clp3", _rem))
        _rem //= 8
    if _rem == 2:
        _PASSES.append(("clp", 2))
    elif _rem == 1:
        _PASSES.append(("j1",))
    _w *= 2


def _log2(x):
    return x.bit_length() - 1


def _make_unpool_kernel():
    mesh = plsc.VectorSubcoreMesh(core_axis_name="c", subcore_axis_name="s")

    @functools.partial(
        pl.kernel,
        mesh=mesh,
        out_type=jax.ShapeDtypeStruct((_ROWS * _H_OUT,), jnp.float32),
        scratch_types=[
            pltpu.VMEM((_H_IN * _L + _L,), jnp.int32),
            pltpu.VMEM((2 * _CHUNK_ROWS * _H_IN,), jnp.int32),
            pltpu.VMEM((2 * _H_IN,), jnp.float32),
            pltpu.VMEM((2 * _H_OUT,), jnp.float32),
            pltpu.SemaphoreType.DMA,
            pltpu.SemaphoreType.DMA,
            pltpu.SemaphoreType.DMA,
        ],
        compiler_params=pltpu.CompilerParams(needs_layout_passes=False),
    )
    def unpool(in_hbm, idx_hbm, out_hbm, packed, chunk, valb, outb,
               sem_idx, sem_val, sem_out):
        wid = lax.axis_index("s") * 2 + lax.axis_index("c")
        row0 = wid * _ROWS_PER_WORKER

        iota = lax.iota(jnp.int32, _L)
        iota16 = iota * _L
        zeros = jnp.zeros((_L,), jnp.float32)
        _CHLEN = _CHUNK_ROWS * _H_IN
        _NCHUNK = _BATCH // _CHUNK_ROWS

        packed[pl.ds(_H_IN * _L, _L)] = jnp.full((_L,), 1 << 25, jnp.int32)

        def batch_body(b, carry):
            base = row0 + b * _BATCH

            pltpu.async_copy(idx_hbm.at[pl.ds(base * _H_IN, _CHLEN)],
                             chunk.at[pl.ds(0, _CHLEN)], sem_idx)

            def chunk_body(ch, c1):
                r0 = base + ch * _CHUNK_ROWS
                cof = (ch & 1) * _CHLEN
                pltpu.make_async_copy(
                    idx_hbm.at[pl.ds(r0 * _H_IN, _CHLEN)],
                    chunk.at[pl.ds(pl.multiple_of(cof, _L), _CHLEN)],
                    sem_idx).wait()

                @pl.when(ch + 1 < _NCHUNK)
                def _():
                    pltpu.async_copy(
                        idx_hbm.at[pl.ds((r0 + _CHUNK_ROWS) * _H_IN, _CHLEN)],
                        chunk.at[pl.ds(pl.multiple_of(
                            ((ch + 1) & 1) * _CHLEN, _L), _CHLEN)],
                        sem_idx)

                def lrow_body(l, c2):
                    l16 = ch * _CHUNK_ROWS + l

                    def tcol_body(c):
                        a = chunk[pl.ds(cof + l * _H_IN + c * _L, _L)]
                        p = (a << 12) | (iota + c * _L)
                        dst = iota16 + (c * (_L * _L) + l16)
                        plsc.store_scatter(packed, [dst], p)

                    plsc.parallel_loop(0, _NVREG, unroll=4)(tcol_body)
                    return c2

                lax.fori_loop(0, _CHUNK_ROWS, lrow_body, 0)
                return c1

            lax.fori_loop(0, _NCHUNK, chunk_body, 0)

            def _ld(i):
                return packed[pl.ds(pl.multiple_of(i, _L), _L)]

            def _st(i, v):
                packed[pl.ds(pl.multiple_of(i, _L), _L)] = v

            def _ce(a, b):
                sw = (a & -4096) > b
                return jnp.where(sw, b, a), jnp.where(sw, a, b)

            for pass_ in _PASSES:
                if pass_[0] == "j1":

                    def j1_body(g):
                        i0 = g * (4 * _L)
                        v0 = _ld(i0)
                        v1 = _ld(i0 + _L)
                        v2 = _ld(i0 + 2 * _L)
                        v3 = _ld(i0 + 3 * _L)
                        v0, v1 = _ce(v0, v1)
                        v2, v3 = _ce(v2, v3)
                        _st(i0, v0)
                        _st(i0 + _L, v1)
                        _st(i0 + 2 * _L, v2)
                        _st(i0 + 3 * _L, v3)

                    plsc.parallel_loop(0, _H_IN // 4, unroll=4)(j1_body)
                elif pass_[0] == "vrc":
                    w = pass_[1]
                    lw = _log2(w)
                    nb = w // 4

                    def vrc_body(g, lw=lw, w=w, nb=nb):
                        blk = (g >> (lw - 2)) << lw
                        off = g & (nb - 1)
                        ia = (blk + off) * _L
                        ib = ia + nb * _L
                        id_ = (blk + (w - 1) - off) * _L
                        ic = id_ - nb * _L
                        va = _ld(ia)
                        vb = _ld(ib)
                        vc = _ld(ic)
                        vd = _ld(id_)
                        va, vd = _ce(va, vd)
                        vb, vc = _ce(vb, vc)
                        va, vb = _ce(va, vb)
                        vc, vd = _ce(vc, vd)
                        _st(ia, va)
                        _st(ib, vb)
                        _st(ic, vc)
                        _st(id_, vd)

                    plsc.parallel_loop(0, _H_IN // 4, unroll=4)(vrc_body)
                elif pass_[0] == "vrc8":
                    w = pass_[1]
                    lw = _log2(w)
                    ne = w // 8

                    def vrc8_body(g, lw=lw, w=w, ne=ne):
                        blk = (g >> (lw - 3)) << lw
                        off = g & (ne - 1)
                        ii = [(blk + off + k * ne) * _L for k in range(4)]
                        mm = [(blk + (w - 1) - off - k * ne) * _L
                              for k in range(4)]
                        a = [_ld(i) for i in ii]
                        m = [_ld(i) for i in mm]
                        for k in range(4):
                            a[k], m[k] = _ce(a[k], m[k])
                        a[0], a[2] = _ce(a[0], a[2])
                        a[1], a[3] = _ce(a[1], a[3])
                        m[3], m[1] = _ce(m[3], m[1])
                        m[2], m[0] = _ce(m[2], m[0])
                        a[0], a[1] = _ce(a[0], a[1])
                        a[2], a[3] = _ce(a[2], a[3])
                        m[3], m[2] = _ce(m[3], m[2])
                        m[1], m[0] = _ce(m[1], m[0])
                        for k in range(4):
                            _st(ii[k], a[k])
                            _st(mm[k], m[k])

                    plsc.parallel_loop(0, _H_IN // 8, unroll=2)(vrc8_body)
                elif pass_[0] == "clp3":
                    j1 = pass_[1]
                    u = j1 // 4
                    lj1 = _log2(j1)

                    def clp3_body(g, lj1=lj1, u=u):
                        blk = (g >> (lj1 - 2)) << (lj1 + 1)
                        off = g & (u - 1)
                        ii = [(blk + off + k * u) * _L for k in range(8)]
                        v = [_ld(i) for i in ii]
                        v[0], v[4] = _ce(v[0], v[4])
                        v[1], v[5] = _ce(v[1], v[5])
                        v[2], v[6] = _ce(v[2], v[6])
                        v[3], v[7] = _ce(v[3], v[7])
                        v[0], v[2] = _ce(v[0], v[2])
                        v[1], v[3] = _ce(v[1], v[3])
                        v[4], v[6] = _ce(v[4], v[6])
                        v[5], v[7] = _ce(v[5], v[7])
                        v[0], v[1] = _ce(v[0], v[1])
                        v[2], v[3] = _ce(v[2], v[3])
                        v[4], v[5] = _ce(v[4], v[5])
                        v[6], v[7] = _ce(v[6], v[7])
                        for k in range(8):
                            _st(ii[k], v[k])

                    plsc.parallel_loop(0, _H_IN // 8, unroll=2)(clp3_body)
                else:
                    j1 = pass_[1]
                    j2 = j1 // 2
                    lj1 = _log2(j1)

                    def clp_body(g, lj1=lj1, j1=j1, j2=j2):
                        blk = (g >> (lj1 - 1)) << (lj1 + 1)
                        off = g & (j2 - 1)
                        i0 = (blk + off) * _L
                        i1 = i0 + j2 * _L
                        i2 = i0 + j1 * _L
                        i3 = i2 + j2 * _L
                        v0 = _ld(i0)
                        v1 = _ld(i1)
                        v2 = _ld(i2)
                        v3 = _ld(i3)
                        v0, v2 = _ce(v0, v2)
                        v1, v3 = _ce(v1, v3)
                        v0, v1 = _ce(v0, v1)
                        v2, v3 = _ce(v2, v3)
                        _st(i0, v0)
                        _st(i1, v1)
                        _st(i2, v2)
                        _st(i3, v3)

                    plsc.parallel_loop(0, _H_IN // 4, unroll=4)(clp_body)

            pltpu.async_copy(in_hbm.at[pl.ds(base * _H_IN, _H_IN)],
                             valb.at[pl.ds(0, _H_IN)], sem_val)

            def out_body(l16, c1):
                row = base + l16
                sl = l16 & 1
                vof = sl * _H_IN
                oof = sl * _H_OUT

                @pl.when(l16 >= 2)
                def _():
                    pltpu.make_async_copy(
                        outb.at[pl.ds(pl.multiple_of(oof, _L), _H_OUT)],
                        out_hbm.at[pl.ds((row - 2) * _H_OUT, _H_OUT)],
                        sem_out).wait()

                @pl.when(l16 + 1 < _BATCH)
                def _():
                    pltpu.async_copy(
                        in_hbm.at[pl.ds((row + 1) * _H_IN, _H_IN)],
                        valb.at[pl.ds(pl.multiple_of((1 - sl) * _H_IN, _L),
                                      _H_IN)],
                        sem_val)

                def zero_body(z):
                    outb[pl.ds(pl.multiple_of(oof + z * _L, _L), _L)] = zeros

                plsc.parallel_loop(0, _H_OUT // _L, unroll=4)(zero_body)

                pltpu.make_async_copy(
                    in_hbm.at[pl.ds(row * _H_IN, _H_IN)],
                    valb.at[pl.ds(pl.multiple_of(vof, _L), _H_IN)],
                    sem_val).wait()

                def scat_body(c):
                    src = iota16 + (c * (_L * _L) + l16)
                    x = plsc.load_gather(packed, [src])
                    nx = plsc.load_gather(packed, [src + _L])
                    key = x >> 12
                    keep = key != (nx >> 12)
                    pos = x & 0xFFF
                    vals = plsc.load_gather(valb, [pos + vof])
                    plsc.store_scatter(outb, [key + oof], vals, mask=keep)

                plsc.parallel_loop(0, _NVREG, unroll=4)(scat_body)
                pltpu.async_copy(
                    outb.at[pl.ds(pl.multiple_of(oof, _L), _H_OUT)],
                    out_hbm.at[pl.ds(row * _H_OUT, _H_OUT)], sem_out)
                return c1

            lax.fori_loop(0, _BATCH, out_body, 0)
            pltpu.make_async_copy(
                outb.at[pl.ds(0, _H_OUT)],
                out_hbm.at[pl.ds((base + _BATCH - 2) * _H_OUT, _H_OUT)],
                sem_out).wait()
            pltpu.make_async_copy(
                outb.at[pl.ds(_H_OUT, _H_OUT)],
                out_hbm.at[pl.ds((base + _BATCH - 1) * _H_OUT, _H_OUT)],
                sem_out).wait()
            return carry

        lax.fori_loop(0, _BATCHES, batch_body, 0)

    return unpool


_unpool = _make_unpool_kernel()


def kernel(input, indices):
    N, C, H_in = input.shape
    flat_in = input.reshape(N * C * H_in)
    flat_idx = indices.reshape(N * C * H_in).astype(jnp.int32)
    out = _unpool(flat_in, flat_idx)
    return out.reshape(N, C, _H_OUT)
